# baseline (speedup 1.0000x reference)
import functools

import jax
import jax.numpy as jnp
from jax import lax
from jax.experimental import pallas as pl
from jax.experimental.pallas import tpu as pltpu
from jax.experimental.pallas import tpu_sc as plsc

NC = 2
NS = 16
NW = NC * NS
LN = 16
CB = 128

_SC_PARAMS = pltpu.CompilerParams(use_tc_tiling_on_sc=False)


def _zero_rows(buf, ncols):
    z = jnp.zeros((LN,), jnp.float32)

    @pl.loop(0, CB)
    def _(b):
        for j in range(ncols // LN):
            buf[b, pl.ds(j * LN, LN)] = z


def _zero_acc_slice(acc, zsrc, row0, nrows):
    full = nrows // CB
    rem = nrows - full * CB

    @pl.loop(0, full)
    def _(i):
        pltpu.sync_copy(zsrc, acc.at[pl.ds(row0 + i * CB, CB)])

    if rem:
        pltpu.sync_copy(zsrc.at[pl.ds(0, rem)],
                        acc.at[pl.ds(row0 + full * CB, rem)])


def _sc_phase1(cd, ei, u, epw):
    n = cd.shape[0]
    chw = epw // CB
    tb = epw - chw * CB
    rpt = n // NS
    mesh = plsc.VectorSubcoreMesh(core_axis_name="c", subcore_axis_name="s")

    @functools.partial(
        pl.kernel,
        out_type=jax.ShapeDtypeStruct((NC, n, 48), jnp.float32),
        mesh=mesh,
        compiler_params=_SC_PARAMS,
        scratch_types=[
            pltpu.VMEM_SHARED((n, 48), jnp.float32),
            pltpu.VMEM_SHARED((n, 64), jnp.float32),
            pltpu.VMEM((epw,), jnp.int32),
            pltpu.VMEM((epw,), jnp.int32),
            pltpu.VMEM((epw,), jnp.float32),
            pltpu.VMEM((CB, 64), jnp.float32),
            pltpu.VMEM((CB, 64), jnp.float32),
            pltpu.VMEM((CB, 48), jnp.float32),
            pltpu.VMEM((CB, 48), jnp.float32),
            pltpu.SemaphoreType.DMA,
            pltpu.SemaphoreType.DMA,
            pltpu.SemaphoreType.DMA,
            pltpu.SemaphoreType.DMA,
        ],
    )
    def k(cd_hbm, ei_hbm, u_hbm, out_hbm,
          acc, cd_sp, srcb, dstb, ub, yrA, yrB, cbA, cbB, gsA, gsB, ssA, ssB):
        c = lax.axis_index("c")
        s = lax.axis_index("s")
        wid = c * NS + s
        row0 = s * rpt
        base = wid * epw

        _zero_rows(cbA, 48)
        _zero_acc_slice(acc, cbA, row0, rpt)
        _zero_rows(cbB, 48)
        one0 = jnp.where(lax.iota(jnp.int32, LN) == 0,
                         jnp.float32(1.0), jnp.float32(0.0))

        @pl.loop(0, CB)
        def _(b):
            cbA[b, pl.ds(32, LN)] = one0
            cbB[b, pl.ds(32, LN)] = one0

        pltpu.sync_copy(ei_hbm.at[0, pl.ds(base, epw)], srcb)
        pltpu.sync_copy(ei_hbm.at[1, pl.ds(base, epw)], dstb)
        pltpu.sync_copy(u_hbm.at[pl.ds(base, epw)], ub)
        pltpu.sync_copy(cd_hbm.at[pl.ds(row0, rpt)], cd_sp.at[pl.ds(row0, rpt)])
        plsc.subcore_barrier()

        def g_start(ch, yr, sem):
            pltpu.async_copy(cd_sp.at[srcb.at[pl.ds(ch * CB, CB)]], yr, sem)

        def g_wait(ch, yr, sem):
            pltpu.make_async_copy(cd_sp.at[srcb.at[pl.ds(ch * CB, CB)]],
                                  yr, sem).wait()

        def s_start(ch, cb, sem):
            pltpu.async_copy(cb, acc.at[dstb.at[pl.ds(ch * CB, CB)]],
                             sem, add=True)

        def s_wait(ch, cb, sem):
            pltpu.make_async_copy(cb, acc.at[dstb.at[pl.ds(ch * CB, CB)]],
                                  sem).wait()

        def compute(ch, yr, cb, ngroups=CB // LN):
            @plsc.parallel_loop(0, ngroups, unroll=4 if ngroups > 4 else 1)
            def _(g):
                u16 = ub[pl.ds(ch * CB + g * LN, LN)]
                for i in range(LN):
                    b = g * LN + i
                    uu = u16[i]
                    for j in range(2):
                        cpart = yr[b, pl.ds(j * LN, LN)]
                        dpart = yr[b, pl.ds(32 + j * LN, LN)]
                        cb[b, pl.ds(j * LN, LN)] = cpart + uu * dpart

        g_start(0, yrA, gsA)
        g_start(1, yrB, gsB)
        g_wait(0, yrA, gsA)
        compute(0, yrA, cbA)
        s_start(0, cbA, ssA)
        g_start(2, yrA, gsA)
        g_wait(1, yrB, gsB)
        compute(1, yrB, cbB)
        s_start(1, cbB, ssB)
        g_start(3, yrB, gsB)

        @pl.loop(2, chw - 2, step=2)
        def _(ch):
            g_wait(ch, yrA, gsA)
            s_wait(ch - 2, cbA, ssA)
            compute(ch, yrA, cbA)
            s_start(ch, cbA, ssA)
            g_start(ch + 2, yrA, gsA)
            g_wait(ch + 1, yrB, gsB)
            s_wait(ch - 1, cbB, ssB)
            compute(ch + 1, yrB, cbB)
            s_start(ch + 1, cbB, ssB)
            g_start(ch + 3, yrB, gsB)

        g_wait(chw - 2, yrA, gsA)
        s_wait(chw - 4, cbA, ssA)
        compute(chw - 2, yrA, cbA)
        s_start(chw - 2, cbA, ssA)
        g_wait(chw - 1, yrB, gsB)
        s_wait(chw - 3, cbB, ssB)
        compute(chw - 1, yrB, cbB)
        s_start(chw - 1, cbB, ssB)
        s_wait(chw - 2, cbA, ssA)
        s_wait(chw - 1, cbB, ssB)

        if tb:
            toff = chw * CB
            pltpu.sync_copy(cd_sp.at[srcb.at[pl.ds(toff, tb)]],
                            yrA.at[pl.ds(0, tb)])

            @pl.loop(0, tb // LN)
            def _(g):
                u16 = ub[pl.ds(toff + g * LN, LN)]
                for i in range(LN):
                    b = g * LN + i
                    uu = u16[i]
                    for j in range(2):
                        cpart = yrA[b, pl.ds(j * LN, LN)]
                        dpart = yrA[b, pl.ds(32 + j * LN, LN)]
                        cbA[b, pl.ds(j * LN, LN)] = cpart + uu * dpart

            pltpu.sync_copy(cbA.at[pl.ds(0, tb)],
                            acc.at[dstb.at[pl.ds(toff, tb)]], add=True)

        plsc.subcore_barrier()
        pltpu.sync_copy(acc.at[pl.ds(row0, rpt)],
                        out_hbm.at[c, pl.ds(row0, rpt)])

    return k(cd, ei, u)


def _sc_phase2(h, ei, u, epw):
    n = h.shape[0]
    chw = epw // CB
    tb = epw - chw * CB
    rpt = n // NS
    mesh = plsc.VectorSubcoreMesh(core_axis_name="c", subcore_axis_name="s")

    @functools.partial(
        pl.kernel,
        out_type=jax.ShapeDtypeStruct((NC, n, 64), jnp.float32),
        mesh=mesh,
        compiler_params=_SC_PARAMS,
        scratch_types=[
            pltpu.VMEM_SHARED((n, 64), jnp.float32),
            pltpu.VMEM_SHARED((n, 32), jnp.float32),
            pltpu.VMEM((epw,), jnp.int32),
            pltpu.VMEM((epw,), jnp.int32),
            pltpu.VMEM((epw,), jnp.float32),
            pltpu.VMEM((CB, 32), jnp.float32),
            pltpu.VMEM((CB, 32), jnp.float32),
            pltpu.VMEM((CB, 64), jnp.float32),
            pltpu.VMEM((CB, 64), jnp.float32),
            pltpu.SemaphoreType.DMA,
            pltpu.SemaphoreType.DMA,
            pltpu.SemaphoreType.DMA,
            pltpu.SemaphoreType.DMA,
        ],
    )
    def k(h_hbm, ei_hbm, u_hbm, o2_hbm,
          acc2, h_sp, srcb, dstb, ub, hrA, hrB, cbA, cbB,
          gsA, gsB, ssA, ssB):
        c = lax.axis_index("c")
        s = lax.axis_index("s")
        wid = c * NS + s
        row0 = s * rpt
        base = wid * epw

        _zero_rows(cbA, 64)
        _zero_acc_slice(acc2, cbA, row0, rpt)

        pltpu.sync_copy(ei_hbm.at[0, pl.ds(base, epw)], srcb)
        pltpu.sync_copy(ei_hbm.at[1, pl.ds(base, epw)], dstb)
        pltpu.sync_copy(u_hbm.at[pl.ds(base, epw)], ub)
        pltpu.sync_copy(h_hbm.at[pl.ds(row0, rpt)], h_sp.at[pl.ds(row0, rpt)])
        plsc.subcore_barrier()

        def g_start(ch, hr, sem):
            pltpu.async_copy(h_sp.at[srcb.at[pl.ds(ch * CB, CB)]], hr, sem)

        def g_wait(ch, hr, sem):
            pltpu.make_async_copy(h_sp.at[srcb.at[pl.ds(ch * CB, CB)]],
                                  hr, sem).wait()

        def s_start(ch, cb, sem):
            pltpu.async_copy(cb, acc2.at[dstb.at[pl.ds(ch * CB, CB)]],
                             sem, add=True)

        def s_wait(ch, cb, sem):
            pltpu.make_async_copy(cb, acc2.at[dstb.at[pl.ds(ch * CB, CB)]],
                                  sem).wait()

        def compute(ch, hr, cb):
            @plsc.parallel_loop(0, CB // LN, unroll=4)
            def _(g):
                u16 = ub[pl.ds(ch * CB + g * LN, LN)]
                for i in range(LN):
                    b = g * LN + i
                    uu = u16[i]
                    for j in range(2):
                        hh = hr[b, pl.ds(j * LN, LN)]
                        cb[b, pl.ds(j * LN, LN)] = hh
                        cb[b, pl.ds(32 + j * LN, LN)] = uu * hh

        g_start(0, hrA, gsA)
        g_start(1, hrB, gsB)
        g_wait(0, hrA, gsA)
        compute(0, hrA, cbA)
        s_start(0, cbA, ssA)
        g_start(2, hrA, gsA)
        g_wait(1, hrB, gsB)
        compute(1, hrB, cbB)
        s_start(1, cbB, ssB)
        g_start(3, hrB, gsB)

        @pl.loop(2, chw - 2, step=2)
        def _(ch):
            g_wait(ch, hrA, gsA)
            s_wait(ch - 2, cbA, ssA)
            compute(ch, hrA, cbA)
            s_start(ch, cbA, ssA)
            g_start(ch + 2, hrA, gsA)
            g_wait(ch + 1, hrB, gsB)
            s_wait(ch - 1, cbB, ssB)
            compute(ch + 1, hrB, cbB)
            s_start(ch + 1, cbB, ssB)
            g_start(ch + 3, hrB, gsB)

        g_wait(chw - 2, hrA, gsA)
        s_wait(chw - 4, cbA, ssA)
        compute(chw - 2, hrA, cbA)
        s_start(chw - 2, cbA, ssA)
        g_wait(chw - 1, hrB, gsB)
        s_wait(chw - 3, cbB, ssB)
        compute(chw - 1, hrB, cbB)
        s_start(chw - 1, cbB, ssB)
        s_wait(chw - 2, cbA, ssA)
        s_wait(chw - 1, cbB, ssB)

        if tb:
            toff = chw * CB
            pltpu.sync_copy(h_sp.at[srcb.at[pl.ds(toff, tb)]],
                            hrA.at[pl.ds(0, tb)])

            @pl.loop(0, tb // LN)
            def _(g):
                u16 = ub[pl.ds(toff + g * LN, LN)]
                for i in range(LN):
                    b = g * LN + i
                    uu = u16[i]
                    for j in range(2):
                        hh = hrA[b, pl.ds(j * LN, LN)]
                        cbA[b, pl.ds(j * LN, LN)] = hh
                        cbA[b, pl.ds(32 + j * LN, LN)] = uu * hh

            pltpu.sync_copy(cbA.at[pl.ds(0, tb)],
                            acc2.at[dstb.at[pl.ds(toff, tb)]], add=True)

        plsc.subcore_barrier()
        pltpu.sync_copy(acc2.at[pl.ds(row0, rpt)],
                        o2_hbm.at[c, pl.ds(row0, rpt)])

    return k(h, ei, u)


def _tc_pre_body(x_ref, w1_ref, r1_ref, cd_ref, xr_ref):
    x = x_ref[...]
    w0 = w1_ref[0]
    wd = w1_ref[1] - w0
    cd_ref[...] = jnp.concatenate(
        [jnp.dot(x, w0), jnp.dot(x, wd)], axis=1)
    xr_ref[...] = jnp.dot(x, r1_ref[...])


def _tc_pre(x, W1, root1):
    n, din = x.shape
    dh = W1.shape[2]
    rb = n // 8 if n % 64 == 0 else n
    return pl.pallas_call(
        _tc_pre_body,
        grid=(n // rb,),
        in_specs=[
            pl.BlockSpec((rb, din), lambda i: (i, 0)),
            pl.BlockSpec((2, din, dh), lambda i: (0, 0, 0)),
            pl.BlockSpec((din, dh), lambda i: (0, 0)),
        ],
        out_specs=[
            pl.BlockSpec((rb, 2 * dh), lambda i: (i, 0)),
            pl.BlockSpec((rb, dh), lambda i: (i, 0)),
        ],
        out_shape=[
            jax.ShapeDtypeStruct((n, 2 * dh), jnp.float32),
            jax.ShapeDtypeStruct((n, dh), jnp.float32),
        ],
    )(x, W1, root1)


def _tc_mid_body(acc_ref, xr_ref, b1_ref, h_ref, rc_ref):
    s = acc_ref[0] + acc_ref[1]
    cnt = s[:, 32:33]
    rc = 1.0 / jnp.maximum(cnt, 1.0)
    h_ref[...] = jnp.maximum(s[:, :32] * rc + xr_ref[...] + b1_ref[...], 0.0)
    rc_ref[...] = rc


def _tc_mid(acc1, xr, b1):
    n = xr.shape[0]
    dh = xr.shape[1]
    rb = n // 8 if n % 64 == 0 else n
    return pl.pallas_call(
        _tc_mid_body,
        grid=(n // rb,),
        in_specs=[
            pl.BlockSpec((NC, rb, 48), lambda i: (0, i, 0)),
            pl.BlockSpec((rb, dh), lambda i: (i, 0)),
            pl.BlockSpec((1, dh), lambda i: (0, 0)),
        ],
        out_specs=[
            pl.BlockSpec((rb, dh), lambda i: (i, 0)),
            pl.BlockSpec((rb, 1), lambda i: (i, 0)),
        ],
        out_shape=[
            jax.ShapeDtypeStruct((n, dh), jnp.float32),
            jax.ShapeDtypeStruct((n, 1), jnp.float32),
        ],
    )(acc1, xr, b1)


def _tc_post_body(a2_ref, rc_ref, h_ref, w2_ref, r2_ref, b2_ref,
                  out_ref):
    s = a2_ref[0] + a2_ref[1]
    s1 = s[:, :32]
    su = s[:, 32:]
    w0 = w2_ref[0]
    wd = w2_ref[1] - w0
    agg = (jnp.dot(s1, w0)
           + jnp.dot(su, wd)) * rc_ref[...]
    z = agg + jnp.dot(h_ref[...], r2_ref[...]) + b2_ref[...]
    m = jnp.max(z, axis=1, keepdims=True)
    zs = z - m
    out_ref[...] = zs - jnp.log(jnp.sum(jnp.exp(zs), axis=1, keepdims=True))


def _tc_post(a2, rc, h, W2, root2, b2):
    n, dh = h.shape
    dout = W2.shape[2]
    rb = n // 8 if n % 64 == 0 else n
    return pl.pallas_call(
        _tc_post_body,
        grid=(n // rb,),
        in_specs=[
            pl.BlockSpec((NC, rb, 2 * dh), lambda i: (0, i, 0)),
            pl.BlockSpec((rb, 1), lambda i: (i, 0)),
            pl.BlockSpec((rb, dh), lambda i: (i, 0)),
            pl.BlockSpec((2, dh, dout), lambda i: (0, 0, 0)),
            pl.BlockSpec((dh, dout), lambda i: (0, 0)),
            pl.BlockSpec((1, dout), lambda i: (0, 0)),
        ],
        out_specs=pl.BlockSpec((rb, dout), lambda i: (i, 0)),
        out_shape=jax.ShapeDtypeStruct((n, dout), jnp.float32),
    )(a2, rc, h, W2, root2, b2)


def kernel(x, edge_index, edge_attr, W1, root1, b1, W2, root2, b2):
    n = x.shape[0]
    e = edge_index.shape[1]
    epw = e // NW

    npad = (-n) % (NS * 8)
    xp = jnp.pad(x, ((0, npad), (0, 0))) if npad else x

    u = edge_attr.reshape(e)

    cd, xr = _tc_pre(xp, W1, root1)
    acc1 = _sc_phase1(cd, edge_index, u, epw)
    h, rc = _tc_mid(acc1, xr, b1.reshape(1, -1))
    acc2 = _sc_phase2(h, edge_index, u, epw)
    out = _tc_post(acc2, rc, h, W2, root2, b2.reshape(1, -1))
    return out[:n] if npad else out


# speedup vs baseline: 1.0180x; 1.0180x over previous
import functools

import jax
import jax.numpy as jnp
from jax import lax
from jax.experimental import pallas as pl
from jax.experimental.pallas import tpu as pltpu
from jax.experimental.pallas import tpu_sc as plsc

NC = 2
NS = 16
NW = NC * NS
LN = 16
CB = 128

_SC_PARAMS = pltpu.CompilerParams(use_tc_tiling_on_sc=False)


def _zero_rows(buf, ncols):
    z = jnp.zeros((LN,), jnp.float32)

    @pl.loop(0, CB)
    def _(b):
        for j in range(ncols // LN):
            buf[b, pl.ds(j * LN, LN)] = z


def _zero_acc_slice(acc, zsrc, row0, nrows):
    full = nrows // CB
    rem = nrows - full * CB

    @pl.loop(0, full)
    def _(i):
        pltpu.sync_copy(zsrc, acc.at[pl.ds(row0 + i * CB, CB)])

    if rem:
        pltpu.sync_copy(zsrc.at[pl.ds(0, rem)],
                        acc.at[pl.ds(row0 + full * CB, rem)])


def _sc_phase1(cd, ei, u, epw):
    n = cd.shape[0]
    chw = epw // CB
    tb = epw - chw * CB
    rpt = n // NS
    mesh = plsc.VectorSubcoreMesh(core_axis_name="c", subcore_axis_name="s")

    @functools.partial(
        pl.kernel,
        out_type=jax.ShapeDtypeStruct((NC, n, 48), jnp.float32),
        mesh=mesh,
        compiler_params=_SC_PARAMS,
        scratch_types=[
            pltpu.VMEM_SHARED((n, 48), jnp.float32),
            pltpu.VMEM_SHARED((n, 64), jnp.float32),
            pltpu.VMEM((epw,), jnp.int32),
            pltpu.VMEM((epw,), jnp.int32),
            pltpu.VMEM((epw,), jnp.float32),
            pltpu.VMEM((CB, 64), jnp.float32),
            pltpu.VMEM((CB, 64), jnp.float32),
            pltpu.VMEM((CB, 48), jnp.float32),
            pltpu.VMEM((CB, 48), jnp.float32),
            pltpu.SemaphoreType.DMA,
            pltpu.SemaphoreType.DMA,
            pltpu.SemaphoreType.DMA,
            pltpu.SemaphoreType.DMA,
        ],
    )
    def k(cd_hbm, ei_hbm, u_hbm, out_hbm,
          acc, cd_sp, srcb, dstb, ub, yrA, yrB, cbA, cbB, gsA, gsB, ssA, ssB):
        c = lax.axis_index("c")
        s = lax.axis_index("s")
        wid = c * NS + s
        row0 = s * rpt
        base = wid * epw

        _zero_rows(cbA, 48)
        _zero_acc_slice(acc, cbA, row0, rpt)
        _zero_rows(cbB, 48)
        one0 = jnp.where(lax.iota(jnp.int32, LN) == 0,
                         jnp.float32(1.0), jnp.float32(0.0))

        @pl.loop(0, CB)
        def _(b):
            cbA[b, pl.ds(32, LN)] = one0
            cbB[b, pl.ds(32, LN)] = one0

        pltpu.sync_copy(ei_hbm.at[0, pl.ds(base, epw)], srcb)
        pltpu.sync_copy(ei_hbm.at[1, pl.ds(base, epw)], dstb)
        pltpu.sync_copy(u_hbm.at[pl.ds(base, epw)], ub)
        pltpu.sync_copy(cd_hbm.at[pl.ds(row0, rpt)], cd_sp.at[pl.ds(row0, rpt)])
        plsc.subcore_barrier()

        def g_start(ch, yr, sem):
            pltpu.async_copy(cd_sp.at[srcb.at[pl.ds(ch * CB, CB)]], yr, sem)

        def g_wait(ch, yr, sem):
            pltpu.make_async_copy(cd_sp.at[srcb.at[pl.ds(ch * CB, CB)]],
                                  yr, sem).wait()

        def s_start(ch, cb, sem):
            pltpu.async_copy(cb, acc.at[dstb.at[pl.ds(ch * CB, CB)]],
                             sem, add=True)

        def s_wait(ch, cb, sem):
            pltpu.make_async_copy(cb, acc.at[dstb.at[pl.ds(ch * CB, CB)]],
                                  sem).wait()

        def compute(ch, yr, cb, ngroups=CB // LN):
            @plsc.parallel_loop(0, ngroups, unroll=4 if ngroups > 4 else 1)
            def _(g):
                u16 = ub[pl.ds(ch * CB + g * LN, LN)]
                for i in range(LN):
                    b = g * LN + i
                    uu = u16[i]
                    for j in range(2):
                        cpart = yr[b, pl.ds(j * LN, LN)]
                        dpart = yr[b, pl.ds(32 + j * LN, LN)]
                        cb[b, pl.ds(j * LN, LN)] = cpart + uu * dpart

        g_start(0, yrA, gsA)
        g_start(1, yrB, gsB)
        g_wait(0, yrA, gsA)
        compute(0, yrA, cbA)
        s_start(0, cbA, ssA)
        g_start(2, yrA, gsA)
        g_wait(1, yrB, gsB)
        compute(1, yrB, cbB)
        s_start(1, cbB, ssB)
        g_start(3, yrB, gsB)

        @pl.loop(2, chw - 2, step=2)
        def _(ch):
            g_wait(ch, yrA, gsA)
            s_wait(ch - 2, cbA, ssA)
            compute(ch, yrA, cbA)
            s_start(ch, cbA, ssA)
            g_start(ch + 2, yrA, gsA)
            g_wait(ch + 1, yrB, gsB)
            s_wait(ch - 1, cbB, ssB)
            compute(ch + 1, yrB, cbB)
            s_start(ch + 1, cbB, ssB)
            g_start(ch + 3, yrB, gsB)

        g_wait(chw - 2, yrA, gsA)
        s_wait(chw - 4, cbA, ssA)
        compute(chw - 2, yrA, cbA)
        s_start(chw - 2, cbA, ssA)
        g_wait(chw - 1, yrB, gsB)
        s_wait(chw - 3, cbB, ssB)
        compute(chw - 1, yrB, cbB)
        s_start(chw - 1, cbB, ssB)
        s_wait(chw - 2, cbA, ssA)
        s_wait(chw - 1, cbB, ssB)

        if tb:
            toff = chw * CB
            pltpu.sync_copy(cd_sp.at[srcb.at[pl.ds(toff, tb)]],
                            yrA.at[pl.ds(0, tb)])

            @pl.loop(0, tb // LN)
            def _(g):
                u16 = ub[pl.ds(toff + g * LN, LN)]
                for i in range(LN):
                    b = g * LN + i
                    uu = u16[i]
                    for j in range(2):
                        cpart = yrA[b, pl.ds(j * LN, LN)]
                        dpart = yrA[b, pl.ds(32 + j * LN, LN)]
                        cbA[b, pl.ds(j * LN, LN)] = cpart + uu * dpart

            pltpu.sync_copy(cbA.at[pl.ds(0, tb)],
                            acc.at[dstb.at[pl.ds(toff, tb)]], add=True)

        plsc.subcore_barrier()
        pltpu.sync_copy(acc.at[pl.ds(row0, rpt)],
                        out_hbm.at[c, pl.ds(row0, rpt)])

    return k(cd, ei, u)


def _sc_phase2(h, ei, u, epw):
    n = h.shape[0]
    chw = epw // CB
    tb = epw - chw * CB
    rpt = n // NS
    mesh = plsc.VectorSubcoreMesh(core_axis_name="c", subcore_axis_name="s")

    @functools.partial(
        pl.kernel,
        out_type=[jax.ShapeDtypeStruct((NC, n, 32), jnp.float32),
                  jax.ShapeDtypeStruct((NC, n, 32), jnp.float32)],
        mesh=mesh,
        compiler_params=_SC_PARAMS,
        scratch_types=[
            pltpu.VMEM_SHARED((n, 32), jnp.float32),
            pltpu.VMEM_SHARED((n, 32), jnp.float32),
            pltpu.VMEM_SHARED((n, 32), jnp.float32),
            pltpu.VMEM((epw,), jnp.int32),
            pltpu.VMEM((epw,), jnp.int32),
            pltpu.VMEM((epw,), jnp.float32),
        ] + [pltpu.VMEM((CB, 32), jnp.float32)] * 8
          + [pltpu.SemaphoreType.DMA] * 8,
    )
    def k(h_hbm, ei_hbm, u_hbm, o1_hbm, ou_hbm,
          acc1, accu, h_sp, srcb, dstb, ub,
          hr0, hr1, hr2, hr3, uh0, uh1, uh2, uh3,
          gs0, gs1, gs2, gs3, ss0, ss1, ss2, ss3):
        c = lax.axis_index("c")
        s = lax.axis_index("s")
        wid = c * NS + s
        row0 = s * rpt
        base = wid * epw
        hr = [hr0, hr1, hr2, hr3]
        uh = [uh0, uh1, uh2, uh3]
        gs = [gs0, gs1, gs2, gs3]
        ss = [ss0, ss1, ss2, ss3]

        _zero_rows(hr0, 32)
        _zero_acc_slice(acc1, hr0, row0, rpt)
        _zero_acc_slice(accu, hr0, row0, rpt)

        pltpu.sync_copy(ei_hbm.at[0, pl.ds(base, epw)], srcb)
        pltpu.sync_copy(ei_hbm.at[1, pl.ds(base, epw)], dstb)
        pltpu.sync_copy(u_hbm.at[pl.ds(base, epw)], ub)
        pltpu.sync_copy(h_hbm.at[pl.ds(row0, rpt)], h_sp.at[pl.ds(row0, rpt)])
        plsc.subcore_barrier()

        def g_start(ch, q):
            pltpu.async_copy(h_sp.at[srcb.at[pl.ds(ch * CB, CB)]],
                             hr[q], gs[q])

        def g_wait(ch, q):
            pltpu.make_async_copy(h_sp.at[srcb.at[pl.ds(ch * CB, CB)]],
                                  hr[q], gs[q]).wait()

        def s_start(ch, q):
            idx = dstb.at[pl.ds(ch * CB, CB)]
            pltpu.async_copy(hr[q], acc1.at[idx], ss[q], add=True)
            pltpu.async_copy(uh[q], accu.at[idx], ss[q], add=True)

        def s_wait(ch, q):
            idx = dstb.at[pl.ds(ch * CB, CB)]
            pltpu.make_async_copy(hr[q], acc1.at[idx], ss[q]).wait()
            pltpu.make_async_copy(uh[q], accu.at[idx], ss[q]).wait()

        def compute(ch, q):
            hrq, uhq = hr[q], uh[q]

            @plsc.parallel_loop(0, CB // LN, unroll=4)
            def _(g):
                u16 = ub[pl.ds(ch * CB + g * LN, LN)]
                for i in range(LN):
                    b = g * LN + i
                    uu = u16[i]
                    for j in range(2):
                        uhq[b, pl.ds(j * LN, LN)] = \
                            uu * hrq[b, pl.ds(j * LN, LN)]

        g_start(0, 0)
        g_start(1, 1)
        g_wait(0, 0)
        compute(0, 0)
        s_start(0, 0)
        g_start(2, 2)
        g_wait(1, 1)
        compute(1, 1)
        s_start(1, 1)
        g_start(3, 3)

        @pl.loop(2, chw - 6, step=4)
        def _(bs):
            for t in range(4):
                ch = bs + t
                q = (2 + t) % 4
                qp = t % 4
                g_wait(ch, q)
                compute(ch, q)
                s_start(ch, q)
                s_wait(ch - 2, qp)
                g_start(ch + 2, qp)

        g_wait(chw - 4, (chw - 4) % 4)
        compute(chw - 4, (chw - 4) % 4)
        s_start(chw - 4, (chw - 4) % 4)
        s_wait(chw - 6, (chw - 6) % 4)
        g_start(chw - 2, (chw - 2) % 4)
        g_wait(chw - 3, (chw - 3) % 4)
        compute(chw - 3, (chw - 3) % 4)
        s_start(chw - 3, (chw - 3) % 4)
        s_wait(chw - 5, (chw - 5) % 4)
        g_start(chw - 1, (chw - 1) % 4)
        g_wait(chw - 2, (chw - 2) % 4)
        compute(chw - 2, (chw - 2) % 4)
        s_start(chw - 2, (chw - 2) % 4)
        g_wait(chw - 1, (chw - 1) % 4)
        compute(chw - 1, (chw - 1) % 4)
        s_start(chw - 1, (chw - 1) % 4)
        s_wait(chw - 4, (chw - 4) % 4)
        s_wait(chw - 3, (chw - 3) % 4)
        s_wait(chw - 2, (chw - 2) % 4)
        s_wait(chw - 1, (chw - 1) % 4)

        if tb:
            toff = chw * CB
            q = (chw - 4) % 4
            pltpu.sync_copy(h_sp.at[srcb.at[pl.ds(toff, tb)]],
                            hr[q].at[pl.ds(0, tb)])

            @pl.loop(0, tb // LN)
            def _(g):
                u16 = ub[pl.ds(toff + g * LN, LN)]
                for i in range(LN):
                    b = g * LN + i
                    uu = u16[i]
                    for j in range(2):
                        uh[q][b, pl.ds(j * LN, LN)] = \
                            uu * hr[q][b, pl.ds(j * LN, LN)]

            idx = dstb.at[pl.ds(toff, tb)]
            pltpu.sync_copy(hr[q].at[pl.ds(0, tb)], acc1.at[idx], add=True)
            pltpu.sync_copy(uh[q].at[pl.ds(0, tb)], accu.at[idx], add=True)

        plsc.subcore_barrier()
        pltpu.sync_copy(acc1.at[pl.ds(row0, rpt)],
                        o1_hbm.at[c, pl.ds(row0, rpt)])
        pltpu.sync_copy(accu.at[pl.ds(row0, rpt)],
                        ou_hbm.at[c, pl.ds(row0, rpt)])

    return k(h, ei, u)


def _tc_pre_body(x_ref, w1_ref, r1_ref, cd_ref, xr_ref):
    x = x_ref[...]
    w0 = w1_ref[0]
    wd = w1_ref[1] - w0
    cd_ref[...] = jnp.concatenate(
        [jnp.dot(x, w0), jnp.dot(x, wd)], axis=1)
    xr_ref[...] = jnp.dot(x, r1_ref[...])


def _tc_pre(x, W1, root1):
    n, din = x.shape
    dh = W1.shape[2]
    rb = n // 8 if n % 64 == 0 else n
    return pl.pallas_call(
        _tc_pre_body,
        grid=(n // rb,),
        in_specs=[
            pl.BlockSpec((rb, din), lambda i: (i, 0)),
            pl.BlockSpec((2, din, dh), lambda i: (0, 0, 0)),
            pl.BlockSpec((din, dh), lambda i: (0, 0)),
        ],
        out_specs=[
            pl.BlockSpec((rb, 2 * dh), lambda i: (i, 0)),
            pl.BlockSpec((rb, dh), lambda i: (i, 0)),
        ],
        out_shape=[
            jax.ShapeDtypeStruct((n, 2 * dh), jnp.float32),
            jax.ShapeDtypeStruct((n, dh), jnp.float32),
        ],
    )(x, W1, root1)


def _tc_mid_body(acc_ref, xr_ref, b1_ref, h_ref, rc_ref):
    s = acc_ref[0] + acc_ref[1]
    cnt = s[:, 32:33]
    rc = 1.0 / jnp.maximum(cnt, 1.0)
    h_ref[...] = jnp.maximum(s[:, :32] * rc + xr_ref[...] + b1_ref[...], 0.0)
    rc_ref[...] = rc


def _tc_mid(acc1, xr, b1):
    n = xr.shape[0]
    dh = xr.shape[1]
    rb = n // 8 if n % 64 == 0 else n
    return pl.pallas_call(
        _tc_mid_body,
        grid=(n // rb,),
        in_specs=[
            pl.BlockSpec((NC, rb, 48), lambda i: (0, i, 0)),
            pl.BlockSpec((rb, dh), lambda i: (i, 0)),
            pl.BlockSpec((1, dh), lambda i: (0, 0)),
        ],
        out_specs=[
            pl.BlockSpec((rb, dh), lambda i: (i, 0)),
            pl.BlockSpec((rb, 1), lambda i: (i, 0)),
        ],
        out_shape=[
            jax.ShapeDtypeStruct((n, dh), jnp.float32),
            jax.ShapeDtypeStruct((n, 1), jnp.float32),
        ],
    )(acc1, xr, b1)


def _tc_post_body(a1_ref, au_ref, rc_ref, h_ref, w2_ref, r2_ref, b2_ref,
                  out_ref):
    s1 = a1_ref[0] + a1_ref[1]
    su = au_ref[0] + au_ref[1]
    w0 = w2_ref[0]
    wd = w2_ref[1] - w0
    agg = (jnp.dot(s1, w0)
           + jnp.dot(su, wd)) * rc_ref[...]
    z = agg + jnp.dot(h_ref[...], r2_ref[...]) + b2_ref[...]
    m = jnp.max(z, axis=1, keepdims=True)
    zs = z - m
    out_ref[...] = zs - jnp.log(jnp.sum(jnp.exp(zs), axis=1, keepdims=True))


def _tc_post(a1, au, rc, h, W2, root2, b2):
    n, dh = h.shape
    dout = W2.shape[2]
    rb = n // 8 if n % 64 == 0 else n
    return pl.pallas_call(
        _tc_post_body,
        grid=(n // rb,),
        in_specs=[
            pl.BlockSpec((NC, rb, dh), lambda i: (0, i, 0)),
            pl.BlockSpec((NC, rb, dh), lambda i: (0, i, 0)),
            pl.BlockSpec((rb, 1), lambda i: (i, 0)),
            pl.BlockSpec((rb, dh), lambda i: (i, 0)),
            pl.BlockSpec((2, dh, dout), lambda i: (0, 0, 0)),
            pl.BlockSpec((dh, dout), lambda i: (0, 0)),
            pl.BlockSpec((1, dout), lambda i: (0, 0)),
        ],
        out_specs=pl.BlockSpec((rb, dout), lambda i: (i, 0)),
        out_shape=jax.ShapeDtypeStruct((n, dout), jnp.float32),
    )(a1, au, rc, h, W2, root2, b2)


def kernel(x, edge_index, edge_attr, W1, root1, b1, W2, root2, b2):
    n = x.shape[0]
    e = edge_index.shape[1]
    epw = e // NW

    npad = (-n) % (NS * 8)
    xp = jnp.pad(x, ((0, npad), (0, 0))) if npad else x

    u = edge_attr.reshape(e)

    cd, xr = _tc_pre(xp, W1, root1)
    acc1 = _sc_phase1(cd, edge_index, u, epw)
    h, rc = _tc_mid(acc1, xr, b1.reshape(1, -1))
    s1p, sup = _sc_phase2(h, edge_index, u, epw)
    out = _tc_post(s1p, sup, rc, h, W2, root2, b2.reshape(1, -1))
    return out[:n] if npad else out
